# baseline (speedup 1.0000x reference)
import functools

import jax
import jax.numpy as jnp
from jax import lax
from jax.experimental import pallas as pl
from jax.experimental.pallas import tpu as pltpu
from jax.experimental.pallas import tpu_sc as plsc

N = 10000
NP = 10240
E = 320000
E_PAD = 327680
D = 128
P = 16384
CHUNK = 128
NCH = E_PAD // CHUNK
NCH_CORE = NCH // 2
MAXT = NCH_CORE // 16
RPT = NP // 16

_MESH = plsc.VectorSubcoreMesh(core_axis_name="c", subcore_axis_name="s")


TPS = NCH_CORE // 16

NCH0 = 640
NCH1 = NCH - NCH0
TPS0 = NCH0 // 16
TPS1 = NCH1 // 16
MPH = 40


@functools.partial(
    pl.kernel,
    mesh=_MESH,
    out_type=jax.ShapeDtypeStruct((2, NP, D), jnp.float32),
    scratch_types=[
        pltpu.VMEM((TPS, CHUNK), jnp.int32),
        pltpu.VMEM((CHUNK, D), jnp.float32),
        pltpu.VMEM_SHARED((NP, D), jnp.float32),
        pltpu.SemaphoreType.DMA,
    ],
)
def _deg_kernel(dst_hbm, ones_hbm, zeros_hbm, out_hbm, idx_d, onesbuf, deg,
                sem):
    cid = lax.axis_index("c")
    sid = lax.axis_index("s")
    base = cid * NCH_CORE + sid * TPS
    pltpu.sync_copy(ones_hbm, onesbuf)
    pltpu.sync_copy(zeros_hbm.at[pl.ds(sid * RPT, RPT)],
                    deg.at[pl.ds(sid * RPT, RPT)])
    pltpu.sync_copy(dst_hbm.at[pl.ds(base, TPS)], idx_d)
    plsc.subcore_barrier()

    def body(u, carry):
        cps = [pltpu.async_copy(onesbuf, deg.at[idx_d.at[u * 8 + b]], sem,
                                add=True)
               for b in range(8)]
        for cp in cps:
            cp.wait()
        return carry

    lax.fori_loop(0, TPS // 8, body, 0)
    plsc.subcore_barrier()
    pltpu.sync_copy(deg.at[pl.ds(sid * RPT, RPT)],
                    out_hbm.at[cid].at[pl.ds(sid * RPT, RPT)])


@functools.partial(
    pl.kernel,
    mesh=_MESH,
    out_type=jax.ShapeDtypeStruct((2, NP, D), jnp.float32),
    scratch_types=[
        pltpu.VMEM((MPH, CHUNK), jnp.int32),
        pltpu.VMEM((MPH, CHUNK), jnp.int32),
        pltpu.VMEM((2, CHUNK, D), jnp.float32),
        pltpu.VMEM_SHARED((NP, D), jnp.float32),
        pltpu.SemaphoreType.DMA,
        pltpu.SemaphoreType.DMA,
    ],
)
def _msg_kernel(hhat_hbm, src_hbm, dst_hbm, zeros_hbm, out_hbm,
                idx_s, idx_d, rows, acc, s0, s1):
    cid = lax.axis_index("c")
    sid = lax.axis_index("s")
    pltpu.sync_copy(zeros_hbm.at[pl.ds(sid * RPT, RPT)],
                    acc.at[pl.ds(sid * RPT, RPT)])
    plsc.subcore_barrier()
    sems = (s0, s1)

    def run(tps_c, core_base):
        base = core_base + sid * tps_c
        for phase in range(tps_c // MPH):
            pltpu.sync_copy(src_hbm.at[pl.ds(base + phase * MPH, MPH)],
                            idx_s)
            pltpu.sync_copy(dst_hbm.at[pl.ds(base + phase * MPH, MPH)],
                            idx_d)

            def body(u, carry):
                cps = [pltpu.async_copy(hhat_hbm.at[idx_s.at[u * 2 + b]],
                                        rows.at[b], sems[b])
                       for b in range(2)]
                for b in range(2):
                    cps[b].wait()
                    pltpu.sync_copy(rows.at[b], acc.at[idx_d.at[u * 2 + b]],
                                    add=True)
                return carry

            lax.fori_loop(0, MPH // 2, body, 0)

    @pl.when(cid == 0)
    def _():
        run(TPS0, 0)

    @pl.when(cid == 1)
    def _():
        run(TPS1, NCH0)
    plsc.subcore_barrier()
    pltpu.sync_copy(acc.at[pl.ds(sid * RPT, RPT)],
                    out_hbm.at[cid].at[pl.ds(sid * RPT, RPT)])


_GPT = P // 32


@functools.partial(
    pl.kernel,
    mesh=_MESH,
    out_type=(jax.ShapeDtypeStruct((P, D), jnp.float32),
              jax.ShapeDtypeStruct((P, D), jnp.float32)),
    scratch_types=[
        pltpu.VMEM((CHUNK,), jnp.int32),
        pltpu.VMEM((CHUNK, D), jnp.float32),
        pltpu.SemaphoreType.DMA,
    ],
)
def _gather_kernel(h2_hbm, tf_hbm, gene_hbm, tf_out, gene_out,
                   idx_v, rows, sem):
    cid = lax.axis_index("c")
    sid = lax.axis_index("s")
    wid = sid * 2 + cid
    for idx_hbm, out_hbm in ((tf_hbm, tf_out), (gene_hbm, gene_out)):
        for k in range(_GPT // CHUNK):
            base = wid * _GPT + k * CHUNK
            pltpu.sync_copy(idx_hbm.at[pl.ds(base, CHUNK)], idx_v)
            pltpu.async_copy(h2_hbm.at[idx_v], rows, sem).wait()
            pltpu.sync_copy(rows, out_hbm.at[pl.ds(base, CHUNK)])


_RB = 1024
_RBH = 2048
_EB = 256


def _prep_body(adj_ref, src_ref, dst_ref):
    src_ref[...] = adj_ref[0]
    dst_ref[...] = adj_ref[1]


def _prep_call(adj3):
    return pl.pallas_call(
        _prep_body,
        grid=(NCH // _EB,),
        in_specs=[pl.BlockSpec((2, _EB, CHUNK), lambda i: (0, i, 0))],
        out_specs=[
            pl.BlockSpec((_EB, CHUNK), lambda i: (i, 0)),
            pl.BlockSpec((_EB, CHUNK), lambda i: (i, 0)),
        ],
        out_shape=[
            jax.ShapeDtypeStruct((NCH, CHUNK), jnp.int32),
            jax.ShapeDtypeStruct((NCH, CHUNK), jnp.int32),
        ],
    )(adj3)


def _ab_body(x_ref, w1_ref, deg_ref, hhat_ref, dinv_ref):
    g = jnp.dot(x_ref[...], w1_ref[...], preferred_element_type=jnp.float32)
    deg = deg_ref[0, :, 0] + deg_ref[1, :, 0] + 1.0
    dinv = lax.rsqrt(deg)[:, None]
    dinv_ref[...] = dinv
    hhat_ref[...] = g * dinv


def _c_body(acc_ref, hhat_ref, dinv_ref, b1_ref, w2_ref, out_ref):
    dinv = dinv_ref[...]
    h1 = jnp.maximum((acc_ref[0] + acc_ref[1] + hhat_ref[...]) * dinv
                     + b1_ref[...], 0.0)
    g2 = jnp.dot(h1, w2_ref[...], preferred_element_type=jnp.float32)
    out_ref[...] = g2 * dinv


def _d_body(acc_ref, hhat_ref, dinv_ref, b2_ref, out_ref):
    out_ref[...] = ((acc_ref[0] + acc_ref[1] + hhat_ref[...])
                    * dinv_ref[...] + b2_ref[...])


def _head_body(tf_ref, gene_ref, w1a_ref, w1b_ref, bh1_ref, w2_ref, bh2_ref,
               out_ref):
    z = (jnp.dot(tf_ref[...], w1a_ref[...], preferred_element_type=jnp.float32)
         + jnp.dot(gene_ref[...], w1b_ref[...],
                   preferred_element_type=jnp.float32)
         + bh1_ref[...])
    g = 0.5 * z * (1.0 + lax.erf(z * 0.7071067811865476))
    out_ref[...] = (jnp.dot(g, w2_ref[...], preferred_element_type=jnp.float32)
                    + bh2_ref[...])


def _ab_call(x_pad, W1, deg2):
    return pl.pallas_call(
        _ab_body,
        grid=(NP // _RB,),
        in_specs=[
            pl.BlockSpec((_RB, D), lambda i: (i, 0)),
            pl.BlockSpec((D, D), lambda i: (0, 0)),
            pl.BlockSpec((2, _RB, D), lambda i: (0, i, 0)),
        ],
        out_specs=[
            pl.BlockSpec((_RB, D), lambda i: (i, 0)),
            pl.BlockSpec((_RB, 1), lambda i: (i, 0)),
        ],
        out_shape=[
            jax.ShapeDtypeStruct((NP, D), jnp.float32),
            jax.ShapeDtypeStruct((NP, 1), jnp.float32),
        ],
    )(x_pad, W1, deg2)


def _c_call(acc1, hhat1, dinv, b1r, W2):
    return pl.pallas_call(
        _c_body,
        grid=(NP // _RB,),
        in_specs=[
            pl.BlockSpec((2, _RB, D), lambda i: (0, i, 0)),
            pl.BlockSpec((_RB, D), lambda i: (i, 0)),
            pl.BlockSpec((_RB, 1), lambda i: (i, 0)),
            pl.BlockSpec((1, D), lambda i: (0, 0)),
            pl.BlockSpec((D, D), lambda i: (0, 0)),
        ],
        out_specs=pl.BlockSpec((_RB, D), lambda i: (i, 0)),
        out_shape=jax.ShapeDtypeStruct((NP, D), jnp.float32),
    )(acc1, hhat1, dinv, b1r, W2)


def _d_call(acc2, hhat2, dinv, b2r):
    return pl.pallas_call(
        _d_body,
        grid=(NP // _RB,),
        in_specs=[
            pl.BlockSpec((2, _RB, D), lambda i: (0, i, 0)),
            pl.BlockSpec((_RB, D), lambda i: (i, 0)),
            pl.BlockSpec((_RB, 1), lambda i: (i, 0)),
            pl.BlockSpec((1, D), lambda i: (0, 0)),
        ],
        out_specs=pl.BlockSpec((_RB, D), lambda i: (i, 0)),
        out_shape=jax.ShapeDtypeStruct((NP, D), jnp.float32),
    )(acc2, hhat2, dinv, b2r)


def _head_call(tf_emb, gene_emb, Wh1a, Wh1b, bh1r, Wh2p, bh2p):
    return pl.pallas_call(
        _head_body,
        grid=(P // _RBH,),
        in_specs=[
            pl.BlockSpec((_RBH, D), lambda i: (i, 0)),
            pl.BlockSpec((_RBH, D), lambda i: (i, 0)),
            pl.BlockSpec((D, D), lambda i: (0, 0)),
            pl.BlockSpec((D, D), lambda i: (0, 0)),
            pl.BlockSpec((1, D), lambda i: (0, 0)),
            pl.BlockSpec((D, D), lambda i: (0, 0)),
            pl.BlockSpec((1, D), lambda i: (0, 0)),
        ],
        out_specs=pl.BlockSpec((_RBH, D), lambda i: (i, 0)),
        out_shape=jax.ShapeDtypeStruct((P, D), jnp.float32),
    )(tf_emb, gene_emb, Wh1a, Wh1b, bh1r, Wh2p, bh2p)


def kernel(x, adj, tf_idx, gene_idx, W1, b1, W2, b2, Wh1, bh1, Wh2, bh2):
    pad_idx = jnp.full((E_PAD - E,), N, jnp.int32)
    src = jnp.concatenate([adj[0].astype(jnp.int32), pad_idx])
    dst = jnp.concatenate([adj[1].astype(jnp.int32), pad_idx])
    src = src.reshape(NCH, CHUNK)
    dst = dst.reshape(NCH, CHUNK)
    tf32 = tf_idx.astype(jnp.int32)
    gene32 = gene_idx.astype(jnp.int32)

    x_pad = jnp.zeros((NP, D), jnp.float32).at[:N].set(x)
    zeros_nd = jnp.zeros((NP, D), jnp.float32)
    ones_col = jnp.zeros((CHUNK, D), jnp.float32).at[:, 0].set(1.0)

    b1r = b1.reshape(1, D)
    b2r = b2.reshape(1, D)
    bh1r = bh1.reshape(1, D)
    Wh1a = Wh1[:D]
    Wh1b = Wh1[D:]
    Wh2p = jnp.zeros((D, D), jnp.float32).at[:, :3].set(Wh2)
    bh2p = jnp.zeros((1, D), jnp.float32).at[0, :3].set(bh2)

    deg2 = _deg_kernel(dst, ones_col, zeros_nd)
    hhat1, dinv = _ab_call(x_pad, W1, deg2)
    acc1 = _msg_kernel(hhat1, src, dst, zeros_nd)
    hhat2 = _c_call(acc1, hhat1, dinv, b1r, W2)
    acc2 = _msg_kernel(hhat2, src, dst, zeros_nd)
    h2 = _d_call(acc2, hhat2, dinv, b2r)
    tf_emb, gene_emb = _gather_kernel(h2, tf32, gene32)
    out = _head_call(tf_emb, gene_emb, Wh1a, Wh1b, bh1r, Wh2p, bh2p)
    return out[:, :3]


# speedup vs baseline: 1.1954x; 1.1954x over previous
import functools

import jax
import jax.numpy as jnp
from jax import lax
from jax.experimental import pallas as pl
from jax.experimental.pallas import tpu as pltpu
from jax.experimental.pallas import tpu_sc as plsc

N = 10000
NP = 10240
E = 320000
E_PAD = 327680
D = 128
P = 16384
CHUNK = 128
NCH = E_PAD // CHUNK
NCH_CORE = NCH // 2
MAXT = NCH_CORE // 16
RPT = NP // 16

_MESH = plsc.VectorSubcoreMesh(core_axis_name="c", subcore_axis_name="s")


TPS = NCH_CORE // 16

NCH0 = 1920
NCH1 = NCH - NCH0
TPS0 = NCH0 // 16
TPS1 = NCH1 // 16
MPH = 40


@functools.partial(
    pl.kernel,
    mesh=_MESH,
    out_type=jax.ShapeDtypeStruct((2, NP, D), jnp.float32),
    scratch_types=[
        pltpu.VMEM((TPS, CHUNK), jnp.int32),
        pltpu.VMEM((CHUNK, D), jnp.float32),
        pltpu.VMEM_SHARED((NP, D), jnp.float32),
        pltpu.SemaphoreType.DMA,
    ],
)
def _deg_kernel(dst_hbm, ones_hbm, zeros_hbm, out_hbm, idx_d, onesbuf, deg,
                sem):
    cid = lax.axis_index("c")
    sid = lax.axis_index("s")
    base = cid * NCH_CORE + sid * TPS
    pltpu.sync_copy(ones_hbm, onesbuf)
    pltpu.sync_copy(zeros_hbm.at[pl.ds(sid * RPT, RPT)],
                    deg.at[pl.ds(sid * RPT, RPT)])
    pltpu.sync_copy(dst_hbm.at[pl.ds(base, TPS)], idx_d)
    plsc.subcore_barrier()

    def body(u, carry):
        cps = [pltpu.async_copy(onesbuf, deg.at[idx_d.at[u * 8 + b]], sem,
                                add=True)
               for b in range(8)]
        for cp in cps:
            cp.wait()
        return carry

    lax.fori_loop(0, TPS // 8, body, 0)
    plsc.subcore_barrier()
    pltpu.sync_copy(deg.at[pl.ds(sid * RPT, RPT)],
                    out_hbm.at[cid].at[pl.ds(sid * RPT, RPT)])


@functools.partial(
    pl.kernel,
    mesh=_MESH,
    out_type=jax.ShapeDtypeStruct((2, NP, D), jnp.float32),
    scratch_types=[
        pltpu.VMEM((MPH, CHUNK), jnp.int32),
        pltpu.VMEM((MPH, CHUNK), jnp.int32),
        pltpu.VMEM((2, CHUNK, D), jnp.float32),
        pltpu.VMEM_SHARED((NP, D), jnp.float32),
        pltpu.SemaphoreType.DMA,
        pltpu.SemaphoreType.DMA,
    ],
)
def _msg_kernel(hhat_hbm, src_hbm, dst_hbm, zeros_hbm, out_hbm,
                idx_s, idx_d, rows, acc, s0, s1):
    cid = lax.axis_index("c")
    sid = lax.axis_index("s")
    pltpu.sync_copy(zeros_hbm.at[pl.ds(sid * RPT, RPT)],
                    acc.at[pl.ds(sid * RPT, RPT)])
    plsc.subcore_barrier()
    sems = (s0, s1)

    def run(tps_c, core_base):
        base = core_base + sid * tps_c
        for phase in range(tps_c // MPH):
            pltpu.sync_copy(src_hbm.at[pl.ds(base + phase * MPH, MPH)],
                            idx_s)
            pltpu.sync_copy(dst_hbm.at[pl.ds(base + phase * MPH, MPH)],
                            idx_d)

            def body(u, carry):
                cps = [pltpu.async_copy(hhat_hbm.at[idx_s.at[u * 2 + b]],
                                        rows.at[b], sems[b])
                       for b in range(2)]
                for b in range(2):
                    cps[b].wait()
                    pltpu.sync_copy(rows.at[b], acc.at[idx_d.at[u * 2 + b]],
                                    add=True)
                return carry

            lax.fori_loop(0, MPH // 2, body, 0)

    @pl.when(cid == 0)
    def _():
        run(TPS0, 0)

    @pl.when(cid == 1)
    def _():
        run(TPS1, NCH0)
    plsc.subcore_barrier()
    pltpu.sync_copy(acc.at[pl.ds(sid * RPT, RPT)],
                    out_hbm.at[cid].at[pl.ds(sid * RPT, RPT)])


_GPT = P // 32


@functools.partial(
    pl.kernel,
    mesh=_MESH,
    out_type=(jax.ShapeDtypeStruct((P, D), jnp.float32),
              jax.ShapeDtypeStruct((P, D), jnp.float32)),
    scratch_types=[
        pltpu.VMEM((CHUNK,), jnp.int32),
        pltpu.VMEM((CHUNK, D), jnp.float32),
        pltpu.SemaphoreType.DMA,
    ],
)
def _gather_kernel(h2_hbm, tf_hbm, gene_hbm, tf_out, gene_out,
                   idx_v, rows, sem):
    cid = lax.axis_index("c")
    sid = lax.axis_index("s")
    wid = sid * 2 + cid
    for idx_hbm, out_hbm in ((tf_hbm, tf_out), (gene_hbm, gene_out)):
        for k in range(_GPT // CHUNK):
            base = wid * _GPT + k * CHUNK
            pltpu.sync_copy(idx_hbm.at[pl.ds(base, CHUNK)], idx_v)
            pltpu.async_copy(h2_hbm.at[idx_v], rows, sem).wait()
            pltpu.sync_copy(rows, out_hbm.at[pl.ds(base, CHUNK)])


_RB = 1024
_RBH = 2048
_EB = 256


def _prep_body(adj_ref, src_ref, dst_ref):
    src_ref[...] = adj_ref[0]
    dst_ref[...] = adj_ref[1]


def _prep_call(adj3):
    return pl.pallas_call(
        _prep_body,
        grid=(NCH // _EB,),
        in_specs=[pl.BlockSpec((2, _EB, CHUNK), lambda i: (0, i, 0))],
        out_specs=[
            pl.BlockSpec((_EB, CHUNK), lambda i: (i, 0)),
            pl.BlockSpec((_EB, CHUNK), lambda i: (i, 0)),
        ],
        out_shape=[
            jax.ShapeDtypeStruct((NCH, CHUNK), jnp.int32),
            jax.ShapeDtypeStruct((NCH, CHUNK), jnp.int32),
        ],
    )(adj3)


def _ab_body(x_ref, w1_ref, deg_ref, hhat_ref, dinv_ref):
    g = jnp.dot(x_ref[...], w1_ref[...], preferred_element_type=jnp.float32)
    deg = deg_ref[0, :, 0] + deg_ref[1, :, 0] + 1.0
    dinv = lax.rsqrt(deg)[:, None]
    dinv_ref[...] = dinv
    hhat_ref[...] = g * dinv


def _c_body(acc_ref, hhat_ref, dinv_ref, b1_ref, w2_ref, out_ref):
    dinv = dinv_ref[...]
    h1 = jnp.maximum((acc_ref[0] + acc_ref[1] + hhat_ref[...]) * dinv
                     + b1_ref[...], 0.0)
    g2 = jnp.dot(h1, w2_ref[...], preferred_element_type=jnp.float32)
    out_ref[...] = g2 * dinv


def _d_body(acc_ref, hhat_ref, dinv_ref, b2_ref, out_ref):
    out_ref[...] = ((acc_ref[0] + acc_ref[1] + hhat_ref[...])
                    * dinv_ref[...] + b2_ref[...])


def _head_body(tf_ref, gene_ref, w1a_ref, w1b_ref, bh1_ref, w2_ref, bh2_ref,
               out_ref):
    z = (jnp.dot(tf_ref[...], w1a_ref[...], preferred_element_type=jnp.float32)
         + jnp.dot(gene_ref[...], w1b_ref[...],
                   preferred_element_type=jnp.float32)
         + bh1_ref[...])
    g = 0.5 * z * (1.0 + lax.erf(z * 0.7071067811865476))
    out_ref[...] = (jnp.dot(g, w2_ref[...], preferred_element_type=jnp.float32)
                    + bh2_ref[...])


def _ab_call(x_pad, W1, deg2):
    return pl.pallas_call(
        _ab_body,
        grid=(NP // _RB,),
        in_specs=[
            pl.BlockSpec((_RB, D), lambda i: (i, 0)),
            pl.BlockSpec((D, D), lambda i: (0, 0)),
            pl.BlockSpec((2, _RB, D), lambda i: (0, i, 0)),
        ],
        out_specs=[
            pl.BlockSpec((_RB, D), lambda i: (i, 0)),
            pl.BlockSpec((_RB, 1), lambda i: (i, 0)),
        ],
        out_shape=[
            jax.ShapeDtypeStruct((NP, D), jnp.float32),
            jax.ShapeDtypeStruct((NP, 1), jnp.float32),
        ],
    )(x_pad, W1, deg2)


def _c_call(acc1, hhat1, dinv, b1r, W2):
    return pl.pallas_call(
        _c_body,
        grid=(NP // _RB,),
        in_specs=[
            pl.BlockSpec((2, _RB, D), lambda i: (0, i, 0)),
            pl.BlockSpec((_RB, D), lambda i: (i, 0)),
            pl.BlockSpec((_RB, 1), lambda i: (i, 0)),
            pl.BlockSpec((1, D), lambda i: (0, 0)),
            pl.BlockSpec((D, D), lambda i: (0, 0)),
        ],
        out_specs=pl.BlockSpec((_RB, D), lambda i: (i, 0)),
        out_shape=jax.ShapeDtypeStruct((NP, D), jnp.float32),
    )(acc1, hhat1, dinv, b1r, W2)


def _d_call(acc2, hhat2, dinv, b2r):
    return pl.pallas_call(
        _d_body,
        grid=(NP // _RB,),
        in_specs=[
            pl.BlockSpec((2, _RB, D), lambda i: (0, i, 0)),
            pl.BlockSpec((_RB, D), lambda i: (i, 0)),
            pl.BlockSpec((_RB, 1), lambda i: (i, 0)),
            pl.BlockSpec((1, D), lambda i: (0, 0)),
        ],
        out_specs=pl.BlockSpec((_RB, D), lambda i: (i, 0)),
        out_shape=jax.ShapeDtypeStruct((NP, D), jnp.float32),
    )(acc2, hhat2, dinv, b2r)


def _head_call(tf_emb, gene_emb, Wh1a, Wh1b, bh1r, Wh2p, bh2p):
    return pl.pallas_call(
        _head_body,
        grid=(P // _RBH,),
        in_specs=[
            pl.BlockSpec((_RBH, D), lambda i: (i, 0)),
            pl.BlockSpec((_RBH, D), lambda i: (i, 0)),
            pl.BlockSpec((D, D), lambda i: (0, 0)),
            pl.BlockSpec((D, D), lambda i: (0, 0)),
            pl.BlockSpec((1, D), lambda i: (0, 0)),
            pl.BlockSpec((D, D), lambda i: (0, 0)),
            pl.BlockSpec((1, D), lambda i: (0, 0)),
        ],
        out_specs=pl.BlockSpec((_RBH, D), lambda i: (i, 0)),
        out_shape=jax.ShapeDtypeStruct((P, D), jnp.float32),
    )(tf_emb, gene_emb, Wh1a, Wh1b, bh1r, Wh2p, bh2p)


def kernel(x, adj, tf_idx, gene_idx, W1, b1, W2, b2, Wh1, bh1, Wh2, bh2):
    pad_idx = jnp.full((E_PAD - E,), N, jnp.int32)
    src = jnp.concatenate([adj[0].astype(jnp.int32), pad_idx])
    dst = jnp.concatenate([adj[1].astype(jnp.int32), pad_idx])
    src = src.reshape(NCH, CHUNK)
    dst = dst.reshape(NCH, CHUNK)
    tf32 = tf_idx.astype(jnp.int32)
    gene32 = gene_idx.astype(jnp.int32)

    x_pad = jnp.zeros((NP, D), jnp.float32).at[:N].set(x)
    zeros_nd = jnp.zeros((NP, D), jnp.float32)
    ones_col = jnp.zeros((CHUNK, D), jnp.float32).at[:, 0].set(1.0)

    b1r = b1.reshape(1, D)
    b2r = b2.reshape(1, D)
    bh1r = bh1.reshape(1, D)
    Wh1a = Wh1[:D]
    Wh1b = Wh1[D:]
    Wh2p = jnp.zeros((D, D), jnp.float32).at[:, :3].set(Wh2)
    bh2p = jnp.zeros((1, D), jnp.float32).at[0, :3].set(bh2)

    deg2 = _deg_kernel(dst, ones_col, zeros_nd)
    hhat1, dinv = _ab_call(x_pad, W1, deg2)
    acc1 = _msg_kernel(hhat1, src, dst, zeros_nd)
    hhat2 = _c_call(acc1, hhat1, dinv, b1r, W2)
    acc2 = _msg_kernel(hhat2, src, dst, zeros_nd)
    h2 = _d_call(acc2, hhat2, dinv, b2r)
    tf_emb, gene_emb = _gather_kernel(h2, tf32, gene32)
    out = _head_call(tf_emb, gene_emb, Wh1a, Wh1b, bh1r, Wh2p, bh2p)
    return out[:, :3]
